# Initial kernel scaffold; baseline (speedup 1.0000x reference)
import jax
import jax.numpy as jnp
from jax.experimental import pallas as pl


def kernel(xyz, points, conv_w0, conv_b0, bn_g0, bn_b0, bn_rm0, bn_rv0, conv_w1, conv_b1, bn_g1, bn_b1, bn_rm1, bn_rv1, conv_w2, conv_b2, bn_g2, bn_b2, bn_rm2, bn_rv2):
    raise NotImplementedError("write your pallas kernel here")


# speedup vs baseline: 2.2691x; 2.2691x over previous
import functools

import jax
import jax.numpy as jnp
from jax.experimental import pallas as pl
from jax.experimental.pallas import tpu as pltpu

B = 4
N = 8192
NPOINT = 1024
RADIUS = 0.2
NSAMPLE = 32
D_FEAT = 16
IN_CH = 19
EPS = 1e-5

SUB = 8
NL = N // SUB
CH_PAD = 32
S_BLK = 128


def _fps_kernel(xyz_ref, newxyz_ref):
    x = xyz_ref[:, 0]
    y = xyz_ref[:, 1]
    z = xyz_ref[:, 2]
    row = jax.lax.broadcasted_iota(jnp.int32, (B, SUB, NL), 1)
    col = jax.lax.broadcasted_iota(jnp.int32, (B, SUB, NL), 2)
    flat = row * NL + col

    slot = jax.lax.broadcasted_iota(jnp.int32, (1, 1, NPOINT), 2)

    def body(i, carry):
        dist, far, acc = carry
        sel = flat == far
        cx = jnp.sum(jnp.where(sel, x, 0.0), axis=(1, 2), keepdims=True)
        cy = jnp.sum(jnp.where(sel, y, 0.0), axis=(1, 2), keepdims=True)
        cz = jnp.sum(jnp.where(sel, z, 0.0), axis=(1, 2), keepdims=True)
        cstack = jnp.concatenate([cx, cy, cz], axis=1)
        acc = jnp.where(slot == i, cstack, acc)
        dx = x - cx
        dy = y - cy
        dz = z - cz
        d = dx * dx + (dy * dy + dz * dz)
        dist = jnp.minimum(dist, d)
        m = jnp.max(dist, axis=(1, 2), keepdims=True)
        far = jnp.min(jnp.where(dist == m, flat, N), axis=(1, 2), keepdims=True)
        return dist, far, acc

    dist0 = jnp.full((B, SUB, NL), 1e10, jnp.float32)
    far0 = jnp.zeros((B, 1, 1), jnp.int32)
    acc0 = jnp.zeros((B, 3, NPOINT), jnp.float32)
    _, _, acc = jax.lax.fori_loop(0, NPOINT, body, (dist0, far0, acc0))
    newxyz_ref[...] = acc


def _fps(xyz):
    xyz4 = xyz.reshape(B, 3, SUB, NL)
    return pl.pallas_call(
        _fps_kernel,
        out_shape=jax.ShapeDtypeStruct((B, 3, NPOINT), jnp.float32),
    )(xyz4)


def _ball_query_gather(xyz, points, new_xyz):
    xt = xyz.transpose(0, 2, 1)
    ct = new_xyz.transpose(0, 2, 1)
    dst2 = jnp.sum(xt ** 2, -1)[:, None, :]
    src2 = jnp.sum(ct ** 2, -1)[:, :, None]
    mm = jnp.matmul(ct.astype(jnp.bfloat16), xt.transpose(0, 2, 1).astype(jnp.bfloat16),
                    preferred_element_type=jnp.float32)
    sq = src2 + dst2 - 2.0 * mm
    iota = jnp.broadcast_to(jnp.arange(N, dtype=jnp.int32), (B, NPOINT, N))
    gi = jnp.where(sq > RADIUS ** 2, N, iota)
    gi = jnp.sort(gi, axis=-1)[:, :, :NSAMPLE]
    first = gi[:, :, 0:1]
    gi = jnp.where(gi == N, jnp.broadcast_to(first, gi.shape), gi)
    table = jnp.concatenate(
        [xt, points.transpose(0, 2, 1),
         jnp.zeros((B, N, CH_PAD - IN_CH), jnp.float32)], axis=-1)
    bidx = jnp.arange(B).reshape(B, 1, 1)
    return table[bidx, gi]


def _mlp_kernel(g_ref, sub_ref, w1_ref, w2_ref, w3_ref,
                p1_ref, p2_ref, p3_ref, out_ref):
    g = g_ref[0]
    sub = sub_ref[0]
    x = (g - sub[:, None, :]).reshape(S_BLK * NSAMPLE, CH_PAD)

    def layer(h, w_ref, p_ref):
        mm = jnp.dot(h.astype(jnp.bfloat16), w_ref[...],
                     preferred_element_type=jnp.float32)
        p = p_ref[...]
        hh = (mm + p[0:1] - p[1:2]) / p[2:3] * p[3:4] + p[4:5]
        return jnp.maximum(hh, 0.0)

    h1 = layer(x, w1_ref, p1_ref)
    h2 = layer(h1, w2_ref, p2_ref)
    h3 = layer(h2, w3_ref, p3_ref)
    out_ref[0] = jnp.max(h3.reshape(S_BLK, NSAMPLE, 64), axis=1)


def _mlp_max(grouped, sub, weights, stats):
    w1, w2, w3 = weights
    p1, p2, p3 = stats
    grid = (B, NPOINT // S_BLK)
    return pl.pallas_call(
        _mlp_kernel,
        grid=grid,
        in_specs=[
            pl.BlockSpec((1, S_BLK, NSAMPLE, CH_PAD), lambda b, s: (b, s, 0, 0)),
            pl.BlockSpec((1, S_BLK, CH_PAD), lambda b, s: (b, s, 0)),
            pl.BlockSpec(w1.shape, lambda b, s: (0, 0)),
            pl.BlockSpec(w2.shape, lambda b, s: (0, 0)),
            pl.BlockSpec(w3.shape, lambda b, s: (0, 0)),
            pl.BlockSpec(p1.shape, lambda b, s: (0, 0)),
            pl.BlockSpec(p2.shape, lambda b, s: (0, 0)),
            pl.BlockSpec(p3.shape, lambda b, s: (0, 0)),
        ],
        out_specs=pl.BlockSpec((1, S_BLK, 64), lambda b, s: (b, s, 0)),
        out_shape=jax.ShapeDtypeStruct((B, NPOINT, 64), jnp.float32),
    )(grouped, sub, w1, w2, w3, p1, p2, p3)


def kernel(xyz, points, conv_w0, conv_b0, bn_g0, bn_b0, bn_rm0, bn_rv0,
           conv_w1, conv_b1, bn_g1, bn_b1, bn_rm1, bn_rv1,
           conv_w2, conv_b2, bn_g2, bn_b2, bn_rm2, bn_rv2):
    new_xyz = _fps(xyz)
    grouped = _ball_query_gather(xyz, points, new_xyz)
    sub = jnp.concatenate(
        [new_xyz.transpose(0, 2, 1),
         jnp.zeros((B, NPOINT, CH_PAD - 3), jnp.float32)], axis=-1)

    def prep_w(w, in_ch):
        wm = w[:, :, 0, 0].T
        pad = jnp.zeros((CH_PAD - in_ch, wm.shape[1]), wm.dtype) if wm.shape[0] < CH_PAD else None
        if pad is not None:
            wm = jnp.concatenate([wm, pad], axis=0)
        return wm.astype(jnp.bfloat16)

    w1 = prep_w(conv_w0, IN_CH)
    w2 = conv_w1[:, :, 0, 0].T.astype(jnp.bfloat16)
    w3 = conv_w2[:, :, 0, 0].T.astype(jnp.bfloat16)
    p1 = jnp.stack([conv_b0, bn_rm0, jnp.sqrt(bn_rv0 + EPS), bn_g0, bn_b0])
    p2 = jnp.stack([conv_b1, bn_rm1, jnp.sqrt(bn_rv1 + EPS), bn_g1, bn_b1])
    p3 = jnp.stack([conv_b2, bn_rm2, jnp.sqrt(bn_rv2 + EPS), bn_g2, bn_b2])

    out = _mlp_max(grouped, sub, (w1, w2, w3), (p1, p2, p3))
    return new_xyz, out.transpose(0, 2, 1)


# speedup vs baseline: 9.7214x; 4.2842x over previous
import functools

import jax
import jax.numpy as jnp
from jax import lax
from jax.experimental import pallas as pl
from jax.experimental.pallas import tpu as pltpu
from jax.experimental.pallas import tpu_sc as plsc

B = 4
N = 8192
NPOINT = 1024
RADIUS = 0.2
NSAMPLE = 32
D_FEAT = 16
IN_CH = 19
EPS = 1e-5

SUB = 8
NL = N // SUB
CH_PAD = 32
S_BLK = 128


def _fps_kernel(xyz_ref, newxyz_ref):
    x = xyz_ref[:, 0]
    y = xyz_ref[:, 1]
    z = xyz_ref[:, 2]
    row = jax.lax.broadcasted_iota(jnp.int32, (B, SUB, NL), 1)
    col = jax.lax.broadcasted_iota(jnp.int32, (B, SUB, NL), 2)
    flat = row * NL + col

    slot = jax.lax.broadcasted_iota(jnp.int32, (1, 1, NPOINT), 2)

    def body(i, carry):
        dist, far, acc = carry
        sel = flat == far
        cx = jnp.sum(jnp.where(sel, x, 0.0), axis=(1, 2), keepdims=True)
        cy = jnp.sum(jnp.where(sel, y, 0.0), axis=(1, 2), keepdims=True)
        cz = jnp.sum(jnp.where(sel, z, 0.0), axis=(1, 2), keepdims=True)
        cstack = jnp.concatenate([cx, cy, cz], axis=1)
        acc = jnp.where(slot == i, cstack, acc)
        dx = x - cx
        dy = y - cy
        dz = z - cz
        d = dx * dx + (dy * dy + dz * dz)
        dist = jnp.minimum(dist, d)
        m = jnp.max(dist, axis=(1, 2), keepdims=True)
        far = jnp.min(jnp.where(dist == m, flat, N), axis=(1, 2), keepdims=True)
        return dist, far, acc

    dist0 = jnp.full((B, SUB, NL), 1e10, jnp.float32)
    far0 = jnp.zeros((B, 1, 1), jnp.int32)
    acc0 = jnp.zeros((B, 3, NPOINT), jnp.float32)
    _, _, acc = jax.lax.fori_loop(0, NPOINT, body, (dist0, far0, acc0))
    newxyz_ref[...] = acc


def _fps(xyz):
    xyz4 = xyz.reshape(B, 3, SUB, NL)
    return pl.pallas_call(
        _fps_kernel,
        out_shape=jax.ShapeDtypeStruct((B, 3, NPOINT), jnp.float32),
    )(xyz4)


NW = 32
SCALE = float(2 ** 29)
INV_SCALE = float(2 ** -29)
S_PER_W = (B * NPOINT) // NW
NCHUNK = N // 16
R2 = RADIUS * RADIUS


def _bq_sc_kernel(xb_hbm, cb_hbm, d2_hbm, s2_hbm, table_hbm, out_hbm,
                  xv, yv, zv, d2v, cxv, cyv, czv, s2v, idxbuf, gidx, srow, rows, packed, sem):
    cid = lax.axis_index("c")
    sid = lax.axis_index("s")
    wid = sid * 2 + cid
    b = wid // 8
    s0 = (wid % 8) * S_PER_W

    pltpu.sync_copy(xb_hbm.at[pl.ds((b * 3 + 0) * N, N)], xv)
    pltpu.sync_copy(xb_hbm.at[pl.ds((b * 3 + 1) * N, N)], yv)
    pltpu.sync_copy(xb_hbm.at[pl.ds((b * 3 + 2) * N, N)], zv)
    pltpu.sync_copy(d2_hbm.at[pl.ds(b * N, N)], d2v)
    pltpu.sync_copy(cb_hbm.at[pl.ds((b * 3 + 0) * NPOINT + s0, S_PER_W)], cxv.at[pl.ds(0, S_PER_W)])
    pltpu.sync_copy(cb_hbm.at[pl.ds((b * 3 + 1) * NPOINT + s0, S_PER_W)], cyv.at[pl.ds(0, S_PER_W)])
    pltpu.sync_copy(cb_hbm.at[pl.ds((b * 3 + 2) * NPOINT + s0, S_PER_W)], czv.at[pl.ds(0, S_PER_W)])
    pltpu.sync_copy(s2_hbm.at[pl.ds(b * NPOINT + s0, S_PER_W)], s2v.at[pl.ds(0, S_PER_W)])

    lane = lax.iota(jnp.int32, 16)
    ones = jnp.ones((16,), jnp.int32)
    zeros = jnp.zeros((16,), jnp.int32)

    def per_centroid(s, _):
        cx = cxv[pl.ds(s, 16)][0]
        cy = cyv[pl.ds(s, 16)][0]
        cz = czv[pl.ds(s, 16)][0]
        s2 = s2v[pl.ds(s, 16)][0]

        def body(ck, cnt):
            base = ck * 16
            m1 = xv[pl.ds(base, 16)] * cx
            m2 = yv[pl.ds(base, 16)] * cy
            m3 = zv[pl.ds(base, 16)] * cz
            q1 = (m1 * SCALE).astype(jnp.int32)
            q2 = (m2 * SCALE).astype(jnp.int32)
            q3 = (m3 * SCALE).astype(jnp.int32)
            mm = ((q1 + q2) + q3).astype(jnp.float32) * INV_SCALE
            sq = (s2 + d2v[pl.ds(base, 16)]) - 2.0 * mm
            mask = jnp.logical_not(sq > R2)
            keys = jnp.where(mask, lane, 999)
            sv = plsc.sort_key_val(keys, base + lane)[1]
            total = plsc.all_reduce_population_count(mask)[0]
            pos = jnp.minimum(cnt + lane, 63)
            plsc.store_scatter(idxbuf, [pos], sv, mask=lane < total)
            return cnt + total

        cnt = lax.fori_loop(0, NCHUNK, body, 0)
        first = idxbuf[pl.ds(0, 16)][0]
        goff = b * N
        full = jnp.full((16,), True, jnp.bool_)
        for h in range(2):
            ids = idxbuf[pl.ds(h * 16, 16)]
            ids = jnp.where(lane + h * 16 < cnt, ids, first)
            plsc.store_scatter(gidx, [s * NSAMPLE + h * 16 + lane], ids + goff, mask=full)
        return 0

    lax.fori_loop(0, S_PER_W, per_centroid, 0)

    row0 = (b * NPOINT + s0) * NSAMPLE

    def gather_grp(j, _):
        g0 = j * 128
        for t in range(8):
            gv = gidx[pl.ds(g0 + t * 16, 16)]
            plsc.store_scatter(srow, [t * 16 + lane], gv // 4, mask=jnp.full((16,), True, jnp.bool_))
        pltpu.async_copy(table_hbm.at[srow], rows, sem).wait()

        def extract(r, _):
            off = (gidx[pl.ds(g0 + r, 16)][0] % 4) * CH_PAD
            packed[pl.ds(r * CH_PAD, 16)] = rows[r, pl.ds(off, 16)]
            packed[pl.ds(r * CH_PAD + 16, 16)] = rows[r, pl.ds(off + 16, 16)]
            return 0

        lax.fori_loop(0, 128, extract, 0)
        pltpu.sync_copy(packed, out_hbm.at[pl.ds((row0 + g0) * CH_PAD, 128 * CH_PAD)])
        return 0

    lax.fori_loop(0, (S_PER_W * NSAMPLE) // 128, gather_grp, 0)


def _ball_query_gather(xyz, points, new_xyz):
    xt = xyz.transpose(0, 2, 1)
    ct = new_xyz.transpose(0, 2, 1)
    dst2 = jnp.sum(xt ** 2, -1)
    src2 = jnp.sum(ct ** 2, -1)
    xb = lax.optimization_barrier(xyz.astype(jnp.bfloat16)).astype(jnp.float32)
    cb = lax.optimization_barrier(new_xyz.astype(jnp.bfloat16)).astype(jnp.float32)
    table = jnp.concatenate(
        [xt, points.transpose(0, 2, 1),
         jnp.zeros((B, N, CH_PAD - IN_CH), jnp.float32)], axis=-1).reshape(B * N, CH_PAD)

    grouped = pl.kernel(
        _bq_sc_kernel,
        out_type=jax.ShapeDtypeStruct((B * NPOINT * NSAMPLE * CH_PAD,), jnp.float32),
        mesh=plsc.VectorSubcoreMesh(core_axis_name="c", subcore_axis_name="s"),
        compiler_params=pltpu.CompilerParams(needs_layout_passes=False),
        scratch_types=[
            pltpu.VMEM((N,), jnp.float32),
            pltpu.VMEM((N,), jnp.float32),
            pltpu.VMEM((N,), jnp.float32),
            pltpu.VMEM((N,), jnp.float32),
            pltpu.VMEM((S_PER_W + 16,), jnp.float32),
            pltpu.VMEM((S_PER_W + 16,), jnp.float32),
            pltpu.VMEM((S_PER_W + 16,), jnp.float32),
            pltpu.VMEM((S_PER_W + 16,), jnp.float32),
            pltpu.VMEM((64,), jnp.int32),
            pltpu.VMEM((S_PER_W * NSAMPLE + 16,), jnp.int32),
            pltpu.VMEM((128,), jnp.int32),
            pltpu.VMEM((128, 128), jnp.float32),
            pltpu.VMEM((128 * CH_PAD,), jnp.float32),
            pltpu.SemaphoreType.DMA,
        ],
    )(xb.reshape(-1), cb.reshape(-1), dst2.reshape(-1), src2.reshape(-1),
      table.reshape(B * N // 4, 4 * CH_PAD))
    return grouped.reshape(B, NPOINT, NSAMPLE, CH_PAD)


def _mlp_kernel(g_ref, sub_ref, w1_ref, w2_ref, w3_ref,
                p1_ref, p2_ref, p3_ref, out_ref):
    g = g_ref[0]
    sub = sub_ref[0]
    x = (g - sub[:, None, :]).reshape(S_BLK * NSAMPLE, CH_PAD)

    def layer(h, w_ref, p_ref):
        mm = jnp.dot(h.astype(jnp.bfloat16), w_ref[...],
                     preferred_element_type=jnp.float32)
        p = p_ref[...]
        hh = (mm + p[0:1] - p[1:2]) / p[2:3] * p[3:4] + p[4:5]
        return jnp.maximum(hh, 0.0)

    h1 = layer(x, w1_ref, p1_ref)
    h2 = layer(h1, w2_ref, p2_ref)
    h3 = layer(h2, w3_ref, p3_ref)
    out_ref[0] = jnp.max(h3.reshape(S_BLK, NSAMPLE, 64), axis=1)


def _mlp_max(grouped, sub, weights, stats):
    w1, w2, w3 = weights
    p1, p2, p3 = stats
    grid = (B, NPOINT // S_BLK)
    return pl.pallas_call(
        _mlp_kernel,
        grid=grid,
        in_specs=[
            pl.BlockSpec((1, S_BLK, NSAMPLE, CH_PAD), lambda b, s: (b, s, 0, 0)),
            pl.BlockSpec((1, S_BLK, CH_PAD), lambda b, s: (b, s, 0)),
            pl.BlockSpec(w1.shape, lambda b, s: (0, 0)),
            pl.BlockSpec(w2.shape, lambda b, s: (0, 0)),
            pl.BlockSpec(w3.shape, lambda b, s: (0, 0)),
            pl.BlockSpec(p1.shape, lambda b, s: (0, 0)),
            pl.BlockSpec(p2.shape, lambda b, s: (0, 0)),
            pl.BlockSpec(p3.shape, lambda b, s: (0, 0)),
        ],
        out_specs=pl.BlockSpec((1, S_BLK, 64), lambda b, s: (b, s, 0)),
        out_shape=jax.ShapeDtypeStruct((B, NPOINT, 64), jnp.float32),
    )(grouped, sub, w1, w2, w3, p1, p2, p3)


def kernel(xyz, points, conv_w0, conv_b0, bn_g0, bn_b0, bn_rm0, bn_rv0,
           conv_w1, conv_b1, bn_g1, bn_b1, bn_rm1, bn_rv1,
           conv_w2, conv_b2, bn_g2, bn_b2, bn_rm2, bn_rv2):
    new_xyz = _fps(xyz)
    grouped = _ball_query_gather(xyz, points, new_xyz)
    sub = jnp.concatenate(
        [new_xyz.transpose(0, 2, 1),
         jnp.zeros((B, NPOINT, CH_PAD - 3), jnp.float32)], axis=-1)

    def prep_w(w, in_ch):
        wm = w[:, :, 0, 0].T
        pad = jnp.zeros((CH_PAD - in_ch, wm.shape[1]), wm.dtype) if wm.shape[0] < CH_PAD else None
        if pad is not None:
            wm = jnp.concatenate([wm, pad], axis=0)
        return wm.astype(jnp.bfloat16)

    w1 = prep_w(conv_w0, IN_CH)
    w2 = conv_w1[:, :, 0, 0].T.astype(jnp.bfloat16)
    w3 = conv_w2[:, :, 0, 0].T.astype(jnp.bfloat16)
    p1 = jnp.stack([conv_b0, bn_rm0, jnp.sqrt(bn_rv0 + EPS), bn_g0, bn_b0])
    p2 = jnp.stack([conv_b1, bn_rm1, jnp.sqrt(bn_rv1 + EPS), bn_g1, bn_b1])
    p3 = jnp.stack([conv_b2, bn_rm2, jnp.sqrt(bn_rv2 + EPS), bn_g2, bn_b2])

    out = _mlp_max(grouped, sub, (w1, w2, w3), (p1, p2, p3))
    return new_xyz, out.transpose(0, 2, 1)


# speedup vs baseline: 18.9350x; 1.9478x over previous
import functools

import jax
import jax.numpy as jnp
from jax import lax
from jax.experimental import pallas as pl
from jax.experimental.pallas import tpu as pltpu
from jax.experimental.pallas import tpu_sc as plsc

B = 4
N = 8192
NPOINT = 1024
RADIUS = 0.2
NSAMPLE = 32
D_FEAT = 16
IN_CH = 19
EPS = 1e-5

SUB = 8
NL = N // SUB
CH_PAD = 32
S_BLK = 128


def _fps_kernel(xyz_ref, newxyz_ref):
    x = xyz_ref[:, 0]
    y = xyz_ref[:, 1]
    z = xyz_ref[:, 2]
    row = jax.lax.broadcasted_iota(jnp.int32, (B, SUB, NL), 1)
    col = jax.lax.broadcasted_iota(jnp.int32, (B, SUB, NL), 2)
    flat = row * NL + col

    slot = jax.lax.broadcasted_iota(jnp.int32, (1, 1, NPOINT), 2)

    def body(i, carry):
        dist, far, acc = carry
        sel = flat == far
        cx = jnp.sum(jnp.where(sel, x, 0.0), axis=(1, 2), keepdims=True)
        cy = jnp.sum(jnp.where(sel, y, 0.0), axis=(1, 2), keepdims=True)
        cz = jnp.sum(jnp.where(sel, z, 0.0), axis=(1, 2), keepdims=True)
        cstack = jnp.concatenate([cx, cy, cz], axis=1)
        acc = jnp.where(slot == i, cstack, acc)
        dx = x - cx
        dy = y - cy
        dz = z - cz
        d = dx * dx + (dy * dy + dz * dz)
        dist = jnp.minimum(dist, d)
        m = jnp.max(dist, axis=(1, 2), keepdims=True)
        far = jnp.min(jnp.where(dist == m, flat, N), axis=(1, 2), keepdims=True)
        return dist, far, acc

    dist0 = jnp.full((B, SUB, NL), 1e10, jnp.float32)
    far0 = jnp.zeros((B, 1, 1), jnp.int32)
    acc0 = jnp.zeros((B, 3, NPOINT), jnp.float32)
    _, _, acc = jax.lax.fori_loop(0, NPOINT, body, (dist0, far0, acc0))
    newxyz_ref[...] = acc


def _fps(xyz):
    xyz4 = xyz.reshape(B, 3, SUB, NL)
    return pl.pallas_call(
        _fps_kernel,
        out_shape=jax.ShapeDtypeStruct((B, 3, NPOINT), jnp.float32),
    )(xyz4)


NW = 32
SCALE = float(2 ** 29)
INV_SCALE = float(2 ** -29)
S_PER_W = (B * NPOINT) // NW
NCHUNK = N // 16
R2 = RADIUS * RADIUS


def _bq_sc_kernel(xb_hbm, cb_hbm, d2_hbm, s2_hbm, table_hbm, out_hbm,
                  xv, yv, zv, d2v, cxv, cyv, czv, s2v, idxbuf, gidx, srow, rows, packed, sem):
    cid = lax.axis_index("c")
    sid = lax.axis_index("s")
    wid = sid * 2 + cid
    b = wid // 8
    s0 = (wid % 8) * S_PER_W

    pltpu.sync_copy(xb_hbm.at[pl.ds((b * 3 + 0) * N, N)], xv)
    pltpu.sync_copy(xb_hbm.at[pl.ds((b * 3 + 1) * N, N)], yv)
    pltpu.sync_copy(xb_hbm.at[pl.ds((b * 3 + 2) * N, N)], zv)
    pltpu.sync_copy(d2_hbm.at[pl.ds(b * N, N)], d2v)
    pltpu.sync_copy(cb_hbm.at[pl.ds((b * 3 + 0) * NPOINT + s0, S_PER_W)], cxv.at[pl.ds(0, S_PER_W)])
    pltpu.sync_copy(cb_hbm.at[pl.ds((b * 3 + 1) * NPOINT + s0, S_PER_W)], cyv.at[pl.ds(0, S_PER_W)])
    pltpu.sync_copy(cb_hbm.at[pl.ds((b * 3 + 2) * NPOINT + s0, S_PER_W)], czv.at[pl.ds(0, S_PER_W)])
    pltpu.sync_copy(s2_hbm.at[pl.ds(b * NPOINT + s0, S_PER_W)], s2v.at[pl.ds(0, S_PER_W)])

    lane = lax.iota(jnp.int32, 16)
    ones = jnp.ones((16,), jnp.int32)
    zeros = jnp.zeros((16,), jnp.int32)

    def per_centroid(s, _):
        cx = cxv[pl.ds(s, 16)][0]
        cy = cyv[pl.ds(s, 16)][0]
        cz = czv[pl.ds(s, 16)][0]
        s2 = s2v[pl.ds(s, 16)][0]

        def cond(carry):
            ck, cnt = carry
            return (cnt < NSAMPLE) & (ck < NCHUNK)

        def body(carry):
            ck, cnt = carry
            base = ck * 16
            m1 = xv[pl.ds(base, 16)] * cx
            m2 = yv[pl.ds(base, 16)] * cy
            m3 = zv[pl.ds(base, 16)] * cz
            q1 = (m1 * SCALE).astype(jnp.int32)
            q2 = (m2 * SCALE).astype(jnp.int32)
            q3 = (m3 * SCALE).astype(jnp.int32)
            mm = ((q1 + q2) + q3).astype(jnp.float32) * INV_SCALE
            sq = (s2 + d2v[pl.ds(base, 16)]) - 2.0 * mm
            mask = jnp.logical_not(sq > R2)
            inc = plsc.cumsum(jnp.where(mask, ones, zeros))
            pos = jnp.minimum(cnt + inc - 1, 63)
            plsc.store_scatter(idxbuf, [pos], base + lane, mask=mask)
            return ck + 1, cnt + inc[15]

        _, cnt = lax.while_loop(cond, body, (0, 0))
        first = idxbuf[pl.ds(0, 16)][0]
        goff = b * N
        full = jnp.full((16,), True, jnp.bool_)
        for h in range(2):
            ids = idxbuf[pl.ds(h * 16, 16)]
            ids = jnp.where(lane + h * 16 < cnt, ids, first)
            plsc.store_scatter(gidx, [s * NSAMPLE + h * 16 + lane], ids + goff, mask=full)
        return 0

    lax.fori_loop(0, S_PER_W, per_centroid, 0)

    row0 = (b * NPOINT + s0) * NSAMPLE

    def gather_grp(j, _):
        g0 = j * 128
        for t in range(8):
            gv = gidx[pl.ds(g0 + t * 16, 16)]
            plsc.store_scatter(srow, [t * 16 + lane], gv // 4, mask=jnp.full((16,), True, jnp.bool_))
        pltpu.async_copy(table_hbm.at[srow], rows, sem).wait()

        def extract(r, _):
            off = (gidx[pl.ds(g0 + r, 16)][0] % 4) * CH_PAD
            packed[pl.ds(r * CH_PAD, 16)] = rows[r, pl.ds(off, 16)]
            packed[pl.ds(r * CH_PAD + 16, 16)] = rows[r, pl.ds(off + 16, 16)]
            return 0

        lax.fori_loop(0, 128, extract, 0)
        pltpu.sync_copy(packed, out_hbm.at[pl.ds((row0 + g0) * CH_PAD, 128 * CH_PAD)])
        return 0

    lax.fori_loop(0, (S_PER_W * NSAMPLE) // 128, gather_grp, 0)


def _ball_query_gather(xyz, points, new_xyz):
    xt = xyz.transpose(0, 2, 1)
    ct = new_xyz.transpose(0, 2, 1)
    dst2 = jnp.sum(xt ** 2, -1)
    src2 = jnp.sum(ct ** 2, -1)
    xb = lax.optimization_barrier(xyz.astype(jnp.bfloat16)).astype(jnp.float32)
    cb = lax.optimization_barrier(new_xyz.astype(jnp.bfloat16)).astype(jnp.float32)
    table = jnp.concatenate(
        [xt, points.transpose(0, 2, 1),
         jnp.zeros((B, N, CH_PAD - IN_CH), jnp.float32)], axis=-1).reshape(B * N, CH_PAD)

    grouped = pl.kernel(
        _bq_sc_kernel,
        out_type=jax.ShapeDtypeStruct((B * NPOINT * NSAMPLE * CH_PAD,), jnp.float32),
        mesh=plsc.VectorSubcoreMesh(core_axis_name="c", subcore_axis_name="s"),
        compiler_params=pltpu.CompilerParams(needs_layout_passes=False),
        scratch_types=[
            pltpu.VMEM((N,), jnp.float32),
            pltpu.VMEM((N,), jnp.float32),
            pltpu.VMEM((N,), jnp.float32),
            pltpu.VMEM((N,), jnp.float32),
            pltpu.VMEM((S_PER_W + 16,), jnp.float32),
            pltpu.VMEM((S_PER_W + 16,), jnp.float32),
            pltpu.VMEM((S_PER_W + 16,), jnp.float32),
            pltpu.VMEM((S_PER_W + 16,), jnp.float32),
            pltpu.VMEM((64,), jnp.int32),
            pltpu.VMEM((S_PER_W * NSAMPLE + 16,), jnp.int32),
            pltpu.VMEM((128,), jnp.int32),
            pltpu.VMEM((128, 128), jnp.float32),
            pltpu.VMEM((128 * CH_PAD,), jnp.float32),
            pltpu.SemaphoreType.DMA,
        ],
    )(xb.reshape(-1), cb.reshape(-1), dst2.reshape(-1), src2.reshape(-1),
      table.reshape(B * N // 4, 4 * CH_PAD))
    return grouped.reshape(B, NPOINT, NSAMPLE, CH_PAD)


def _mlp_kernel(g_ref, sub_ref, w1_ref, w2_ref, w3_ref,
                p1_ref, p2_ref, p3_ref, out_ref):
    g = g_ref[0]
    sub = sub_ref[0]
    x = (g - sub[:, None, :]).reshape(S_BLK * NSAMPLE, CH_PAD)

    def layer(h, w_ref, p_ref):
        mm = jnp.dot(h.astype(jnp.bfloat16), w_ref[...],
                     preferred_element_type=jnp.float32)
        p = p_ref[...]
        hh = (mm + p[0:1] - p[1:2]) / p[2:3] * p[3:4] + p[4:5]
        return jnp.maximum(hh, 0.0)

    h1 = layer(x, w1_ref, p1_ref)
    h2 = layer(h1, w2_ref, p2_ref)
    h3 = layer(h2, w3_ref, p3_ref)
    out_ref[0] = jnp.max(h3.reshape(S_BLK, NSAMPLE, 64), axis=1)


def _mlp_max(grouped, sub, weights, stats):
    w1, w2, w3 = weights
    p1, p2, p3 = stats
    grid = (B, NPOINT // S_BLK)
    return pl.pallas_call(
        _mlp_kernel,
        grid=grid,
        in_specs=[
            pl.BlockSpec((1, S_BLK, NSAMPLE, CH_PAD), lambda b, s: (b, s, 0, 0)),
            pl.BlockSpec((1, S_BLK, CH_PAD), lambda b, s: (b, s, 0)),
            pl.BlockSpec(w1.shape, lambda b, s: (0, 0)),
            pl.BlockSpec(w2.shape, lambda b, s: (0, 0)),
            pl.BlockSpec(w3.shape, lambda b, s: (0, 0)),
            pl.BlockSpec(p1.shape, lambda b, s: (0, 0)),
            pl.BlockSpec(p2.shape, lambda b, s: (0, 0)),
            pl.BlockSpec(p3.shape, lambda b, s: (0, 0)),
        ],
        out_specs=pl.BlockSpec((1, S_BLK, 64), lambda b, s: (b, s, 0)),
        out_shape=jax.ShapeDtypeStruct((B, NPOINT, 64), jnp.float32),
    )(grouped, sub, w1, w2, w3, p1, p2, p3)


def kernel(xyz, points, conv_w0, conv_b0, bn_g0, bn_b0, bn_rm0, bn_rv0,
           conv_w1, conv_b1, bn_g1, bn_b1, bn_rm1, bn_rv1,
           conv_w2, conv_b2, bn_g2, bn_b2, bn_rm2, bn_rv2):
    new_xyz = _fps(xyz)
    grouped = _ball_query_gather(xyz, points, new_xyz)
    sub = jnp.concatenate(
        [new_xyz.transpose(0, 2, 1),
         jnp.zeros((B, NPOINT, CH_PAD - 3), jnp.float32)], axis=-1)

    def prep_w(w, in_ch):
        wm = w[:, :, 0, 0].T
        pad = jnp.zeros((CH_PAD - in_ch, wm.shape[1]), wm.dtype) if wm.shape[0] < CH_PAD else None
        if pad is not None:
            wm = jnp.concatenate([wm, pad], axis=0)
        return wm.astype(jnp.bfloat16)

    w1 = prep_w(conv_w0, IN_CH)
    w2 = conv_w1[:, :, 0, 0].T.astype(jnp.bfloat16)
    w3 = conv_w2[:, :, 0, 0].T.astype(jnp.bfloat16)
    p1 = jnp.stack([conv_b0, bn_rm0, jnp.sqrt(bn_rv0 + EPS), bn_g0, bn_b0])
    p2 = jnp.stack([conv_b1, bn_rm1, jnp.sqrt(bn_rv1 + EPS), bn_g1, bn_b1])
    p3 = jnp.stack([conv_b2, bn_rm2, jnp.sqrt(bn_rv2 + EPS), bn_g2, bn_b2])

    out = _mlp_max(grouped, sub, (w1, w2, w3), (p1, p2, p3))
    return new_xyz, out.transpose(0, 2, 1)


# speedup vs baseline: 23.3682x; 1.2341x over previous
import functools

import jax
import jax.numpy as jnp
from jax import lax
from jax.experimental import pallas as pl
from jax.experimental.pallas import tpu as pltpu
from jax.experimental.pallas import tpu_sc as plsc

B = 4
N = 8192
NPOINT = 1024
RADIUS = 0.2
NSAMPLE = 32
D_FEAT = 16
IN_CH = 19
EPS = 1e-5

SUB = 8
NL = N // SUB
CH_PAD = 32
S_BLK = 128


def _fps_kernel(xyz_ref, newxyz_ref):
    x = xyz_ref[:, 0]
    y = xyz_ref[:, 1]
    z = xyz_ref[:, 2]
    row = jax.lax.broadcasted_iota(jnp.int32, (B, SUB, NL), 1)
    col = jax.lax.broadcasted_iota(jnp.int32, (B, SUB, NL), 2)
    flat = row * NL + col

    slot = jax.lax.broadcasted_iota(jnp.int32, (1, 1, NPOINT), 2)

    def body(i, carry):
        dist, far, acc = carry
        sel = flat == far
        cx = jnp.sum(jnp.where(sel, x, 0.0), axis=(1, 2), keepdims=True)
        cy = jnp.sum(jnp.where(sel, y, 0.0), axis=(1, 2), keepdims=True)
        cz = jnp.sum(jnp.where(sel, z, 0.0), axis=(1, 2), keepdims=True)
        cstack = jnp.concatenate([cx, cy, cz], axis=1)
        acc = jnp.where(slot == i, cstack, acc)
        dx = x - cx
        dy = y - cy
        dz = z - cz
        d = dx * dx + (dy * dy + dz * dz)
        dist = jnp.minimum(dist, d)
        m = jnp.max(dist, axis=(1, 2), keepdims=True)
        far = jnp.min(jnp.where(dist == m, flat, N), axis=(1, 2), keepdims=True)
        return dist, far, acc

    dist0 = jnp.full((B, SUB, NL), 1e10, jnp.float32)
    far0 = jnp.zeros((B, 1, 1), jnp.int32)
    acc0 = jnp.zeros((B, 3, NPOINT), jnp.float32)
    _, _, acc = jax.lax.fori_loop(0, NPOINT, body, (dist0, far0, acc0))
    newxyz_ref[...] = acc


def _fps(xyz):
    xyz4 = xyz.reshape(B, 3, SUB, NL)
    return pl.pallas_call(
        _fps_kernel,
        out_shape=jax.ShapeDtypeStruct((B, 3, NPOINT), jnp.float32),
    )(xyz4)


NW = 32
SCALE = float(2 ** 29)
INV_SCALE = float(2 ** -29)
S_PER_W = (B * NPOINT) // NW
NCHUNK = N // 16
R2 = RADIUS * RADIUS


def _bq_sc_kernel(xb_hbm, cb_hbm, d2_hbm, s2_hbm, table_hbm, out_hbm,
                  xv, yv, zv, d2v, cxv, cyv, czv, s2v, idxbuf, gidx, srow, rows, packed, sem):
    cid = lax.axis_index("c")
    sid = lax.axis_index("s")
    wid = sid * 2 + cid
    b = wid // 8
    s0 = (wid % 8) * S_PER_W

    pltpu.sync_copy(xb_hbm.at[pl.ds((b * 3 + 0) * N, N)], xv)
    pltpu.sync_copy(xb_hbm.at[pl.ds((b * 3 + 1) * N, N)], yv)
    pltpu.sync_copy(xb_hbm.at[pl.ds((b * 3 + 2) * N, N)], zv)
    pltpu.sync_copy(d2_hbm.at[pl.ds(b * N, N)], d2v)
    pltpu.sync_copy(cb_hbm.at[pl.ds((b * 3 + 0) * NPOINT + s0, S_PER_W)], cxv.at[pl.ds(0, S_PER_W)])
    pltpu.sync_copy(cb_hbm.at[pl.ds((b * 3 + 1) * NPOINT + s0, S_PER_W)], cyv.at[pl.ds(0, S_PER_W)])
    pltpu.sync_copy(cb_hbm.at[pl.ds((b * 3 + 2) * NPOINT + s0, S_PER_W)], czv.at[pl.ds(0, S_PER_W)])
    pltpu.sync_copy(s2_hbm.at[pl.ds(b * NPOINT + s0, S_PER_W)], s2v.at[pl.ds(0, S_PER_W)])

    lane = lax.iota(jnp.int32, 16)
    ones = jnp.ones((16,), jnp.int32)
    zeros = jnp.zeros((16,), jnp.int32)

    def per_centroid(s, _):
        cx = cxv[pl.ds(s, 16)][0]
        cy = cyv[pl.ds(s, 16)][0]
        cz = czv[pl.ds(s, 16)][0]
        s2 = s2v[pl.ds(s, 16)][0]

        def chunk_mask(base):
            m1 = xv[pl.ds(base, 16)] * cx
            m2 = yv[pl.ds(base, 16)] * cy
            m3 = zv[pl.ds(base, 16)] * cz
            q1 = (m1 * SCALE).astype(jnp.int32)
            q2 = (m2 * SCALE).astype(jnp.int32)
            q3 = (m3 * SCALE).astype(jnp.int32)
            mm = ((q1 + q2) + q3).astype(jnp.float32) * INV_SCALE
            sq = (s2 + d2v[pl.ds(base, 16)]) - 2.0 * mm
            return jnp.logical_not(sq > R2)

        def cond(carry):
            ck, cnt = carry
            return (cnt < NSAMPLE) & (ck < NCHUNK // 2)

        def body(carry):
            ck, cnt = carry
            base = ck * 32
            mask0 = chunk_mask(base)
            mask1 = chunk_mask(base + 16)
            t0 = plsc.all_reduce_population_count(mask0)[0]
            t1 = plsc.all_reduce_population_count(mask1)[0]
            inc0 = plsc.cumsum(jnp.where(mask0, ones, zeros))
            plsc.store_scatter(idxbuf, [jnp.minimum(cnt + inc0 - 1, 63)],
                               base + lane, mask=mask0)
            inc1 = plsc.cumsum(jnp.where(mask1, ones, zeros))
            plsc.store_scatter(idxbuf, [jnp.minimum(cnt + t0 + inc1 - 1, 63)],
                               base + 16 + lane, mask=mask1)
            return ck + 1, cnt + (t0 + t1)

        _, cnt = lax.while_loop(cond, body, (0, 0))
        first = idxbuf[pl.ds(0, 16)][0]
        goff = b * N
        full = jnp.full((16,), True, jnp.bool_)
        for h in range(2):
            ids = idxbuf[pl.ds(h * 16, 16)]
            ids = jnp.where(lane + h * 16 < cnt, ids, first)
            plsc.store_scatter(gidx, [s * NSAMPLE + h * 16 + lane], ids + goff, mask=full)
        return 0

    lax.fori_loop(0, S_PER_W, per_centroid, 0)

    row0 = (b * NPOINT + s0) * NSAMPLE

    def gather_grp(j, _):
        g0 = j * 128
        for t in range(8):
            gv = gidx[pl.ds(g0 + t * 16, 16)]
            plsc.store_scatter(srow, [t * 16 + lane], gv // 4, mask=jnp.full((16,), True, jnp.bool_))
        pltpu.async_copy(table_hbm.at[srow], rows, sem).wait()

        def extract(r, _):
            off = (gidx[pl.ds(g0 + r, 16)][0] % 4) * CH_PAD
            packed[pl.ds(r * CH_PAD, 16)] = rows[r, pl.ds(off, 16)]
            packed[pl.ds(r * CH_PAD + 16, 16)] = rows[r, pl.ds(off + 16, 16)]
            return 0

        lax.fori_loop(0, 128, extract, 0)
        pltpu.sync_copy(packed, out_hbm.at[pl.ds((row0 + g0) * CH_PAD, 128 * CH_PAD)])
        return 0

    lax.fori_loop(0, (S_PER_W * NSAMPLE) // 128, gather_grp, 0)


def _ball_query_gather(xyz, points, new_xyz):
    xt = xyz.transpose(0, 2, 1)
    ct = new_xyz.transpose(0, 2, 1)
    dst2 = jnp.sum(xt ** 2, -1)
    src2 = jnp.sum(ct ** 2, -1)
    xb = lax.optimization_barrier(xyz.astype(jnp.bfloat16)).astype(jnp.float32)
    cb = lax.optimization_barrier(new_xyz.astype(jnp.bfloat16)).astype(jnp.float32)
    table = jnp.concatenate(
        [xt, points.transpose(0, 2, 1),
         jnp.zeros((B, N, CH_PAD - IN_CH), jnp.float32)], axis=-1).reshape(B * N, CH_PAD)

    grouped = pl.kernel(
        _bq_sc_kernel,
        out_type=jax.ShapeDtypeStruct((B * NPOINT * NSAMPLE * CH_PAD,), jnp.float32),
        mesh=plsc.VectorSubcoreMesh(core_axis_name="c", subcore_axis_name="s"),
        compiler_params=pltpu.CompilerParams(needs_layout_passes=False),
        scratch_types=[
            pltpu.VMEM((N,), jnp.float32),
            pltpu.VMEM((N,), jnp.float32),
            pltpu.VMEM((N,), jnp.float32),
            pltpu.VMEM((N,), jnp.float32),
            pltpu.VMEM((S_PER_W + 16,), jnp.float32),
            pltpu.VMEM((S_PER_W + 16,), jnp.float32),
            pltpu.VMEM((S_PER_W + 16,), jnp.float32),
            pltpu.VMEM((S_PER_W + 16,), jnp.float32),
            pltpu.VMEM((64,), jnp.int32),
            pltpu.VMEM((S_PER_W * NSAMPLE + 16,), jnp.int32),
            pltpu.VMEM((128,), jnp.int32),
            pltpu.VMEM((128, 128), jnp.float32),
            pltpu.VMEM((128 * CH_PAD,), jnp.float32),
            pltpu.SemaphoreType.DMA,
        ],
    )(xb.reshape(-1), cb.reshape(-1), dst2.reshape(-1), src2.reshape(-1),
      table.reshape(B * N // 4, 4 * CH_PAD))
    return grouped.reshape(B, NPOINT, NSAMPLE, CH_PAD)


def _mlp_kernel(g_ref, sub_ref, w1_ref, w2_ref, w3_ref,
                p1_ref, p2_ref, p3_ref, out_ref):
    g = g_ref[0]
    sub = sub_ref[0]
    x = (g - sub[:, None, :]).reshape(S_BLK * NSAMPLE, CH_PAD)

    def layer(h, w_ref, p_ref):
        mm = jnp.dot(h.astype(jnp.bfloat16), w_ref[...],
                     preferred_element_type=jnp.float32)
        p = p_ref[...]
        hh = (mm + p[0:1] - p[1:2]) / p[2:3] * p[3:4] + p[4:5]
        return jnp.maximum(hh, 0.0)

    h1 = layer(x, w1_ref, p1_ref)
    h2 = layer(h1, w2_ref, p2_ref)
    h3 = layer(h2, w3_ref, p3_ref)
    out_ref[0] = jnp.max(h3.reshape(S_BLK, NSAMPLE, 64), axis=1)


def _mlp_max(grouped, sub, weights, stats):
    w1, w2, w3 = weights
    p1, p2, p3 = stats
    grid = (B, NPOINT // S_BLK)
    return pl.pallas_call(
        _mlp_kernel,
        grid=grid,
        in_specs=[
            pl.BlockSpec((1, S_BLK, NSAMPLE, CH_PAD), lambda b, s: (b, s, 0, 0)),
            pl.BlockSpec((1, S_BLK, CH_PAD), lambda b, s: (b, s, 0)),
            pl.BlockSpec(w1.shape, lambda b, s: (0, 0)),
            pl.BlockSpec(w2.shape, lambda b, s: (0, 0)),
            pl.BlockSpec(w3.shape, lambda b, s: (0, 0)),
            pl.BlockSpec(p1.shape, lambda b, s: (0, 0)),
            pl.BlockSpec(p2.shape, lambda b, s: (0, 0)),
            pl.BlockSpec(p3.shape, lambda b, s: (0, 0)),
        ],
        out_specs=pl.BlockSpec((1, S_BLK, 64), lambda b, s: (b, s, 0)),
        out_shape=jax.ShapeDtypeStruct((B, NPOINT, 64), jnp.float32),
    )(grouped, sub, w1, w2, w3, p1, p2, p3)


def kernel(xyz, points, conv_w0, conv_b0, bn_g0, bn_b0, bn_rm0, bn_rv0,
           conv_w1, conv_b1, bn_g1, bn_b1, bn_rm1, bn_rv1,
           conv_w2, conv_b2, bn_g2, bn_b2, bn_rm2, bn_rv2):
    new_xyz = _fps(xyz)
    grouped = _ball_query_gather(xyz, points, new_xyz)
    sub = jnp.concatenate(
        [new_xyz.transpose(0, 2, 1),
         jnp.zeros((B, NPOINT, CH_PAD - 3), jnp.float32)], axis=-1)

    def prep_w(w, in_ch):
        wm = w[:, :, 0, 0].T
        pad = jnp.zeros((CH_PAD - in_ch, wm.shape[1]), wm.dtype) if wm.shape[0] < CH_PAD else None
        if pad is not None:
            wm = jnp.concatenate([wm, pad], axis=0)
        return wm.astype(jnp.bfloat16)

    w1 = prep_w(conv_w0, IN_CH)
    w2 = conv_w1[:, :, 0, 0].T.astype(jnp.bfloat16)
    w3 = conv_w2[:, :, 0, 0].T.astype(jnp.bfloat16)
    p1 = jnp.stack([conv_b0, bn_rm0, jnp.sqrt(bn_rv0 + EPS), bn_g0, bn_b0])
    p2 = jnp.stack([conv_b1, bn_rm1, jnp.sqrt(bn_rv1 + EPS), bn_g1, bn_b1])
    p3 = jnp.stack([conv_b2, bn_rm2, jnp.sqrt(bn_rv2 + EPS), bn_g2, bn_b2])

    out = _mlp_max(grouped, sub, (w1, w2, w3), (p1, p2, p3))
    return new_xyz, out.transpose(0, 2, 1)


# speedup vs baseline: 23.8541x; 1.0208x over previous
import functools

import jax
import jax.numpy as jnp
from jax import lax
from jax.experimental import pallas as pl
from jax.experimental.pallas import tpu as pltpu
from jax.experimental.pallas import tpu_sc as plsc

B = 4
N = 8192
NPOINT = 1024
RADIUS = 0.2
NSAMPLE = 32
D_FEAT = 16
IN_CH = 19
EPS = 1e-5

SUB = 8
NL = N // SUB
CH_PAD = 32
S_BLK = 128


def _fps_kernel(xyz_ref, newxyz_ref):
    x = xyz_ref[:, 0]
    y = xyz_ref[:, 1]
    z = xyz_ref[:, 2]
    row = jax.lax.broadcasted_iota(jnp.int32, (B, SUB, NL), 1)
    col = jax.lax.broadcasted_iota(jnp.int32, (B, SUB, NL), 2)
    flat = row * NL + col

    slot = jax.lax.broadcasted_iota(jnp.int32, (1, 1, NPOINT), 2)

    xyzv = xyz_ref[...]

    def body(i, carry):
        dist, far, acc = carry
        sel = flat == far
        csum = jnp.sum(jnp.where(sel[:, None], xyzv, 0.0), axis=(2, 3))
        cstack = csum[:, :, None]
        cx = csum[:, 0:1, None]
        cy = csum[:, 1:2, None]
        cz = csum[:, 2:3, None]
        acc = jnp.where(slot == i, cstack, acc)
        dx = x - cx
        dy = y - cy
        dz = z - cz
        d = dx * dx + (dy * dy + dz * dz)
        dist = jnp.minimum(dist, d)
        m = jnp.max(dist, axis=(1, 2), keepdims=True)
        far = jnp.min(jnp.where(dist == m, flat, N), axis=(1, 2), keepdims=True)
        return dist, far, acc

    dist0 = jnp.full((B, SUB, NL), 1e10, jnp.float32)
    far0 = jnp.zeros((B, 1, 1), jnp.int32)
    acc0 = jnp.zeros((B, 3, NPOINT), jnp.float32)
    _, _, acc = jax.lax.fori_loop(0, NPOINT, body, (dist0, far0, acc0))
    newxyz_ref[...] = acc


def _fps(xyz):
    xyz4 = xyz.reshape(B, 3, SUB, NL)
    return pl.pallas_call(
        _fps_kernel,
        out_shape=jax.ShapeDtypeStruct((B, 3, NPOINT), jnp.float32),
    )(xyz4)


NW = 32
SCALE = float(2 ** 29)
INV_SCALE = float(2 ** -29)
S_PER_W = (B * NPOINT) // NW
NCHUNK = N // 16
R2 = RADIUS * RADIUS


def _bq_sc_kernel(xb_hbm, cb_hbm, d2_hbm, s2_hbm, table_hbm, out_hbm,
                  xv, yv, zv, d2v, cxv, cyv, czv, s2v, idxbuf, gidx, srow, rows, packed, sem):
    cid = lax.axis_index("c")
    sid = lax.axis_index("s")
    wid = sid * 2 + cid
    b = wid // 8
    s0 = (wid % 8) * S_PER_W

    pltpu.sync_copy(xb_hbm.at[pl.ds((b * 3 + 0) * N, N)], xv)
    pltpu.sync_copy(xb_hbm.at[pl.ds((b * 3 + 1) * N, N)], yv)
    pltpu.sync_copy(xb_hbm.at[pl.ds((b * 3 + 2) * N, N)], zv)
    pltpu.sync_copy(d2_hbm.at[pl.ds(b * N, N)], d2v)
    pltpu.sync_copy(cb_hbm.at[pl.ds((b * 3 + 0) * NPOINT + s0, S_PER_W)], cxv.at[pl.ds(0, S_PER_W)])
    pltpu.sync_copy(cb_hbm.at[pl.ds((b * 3 + 1) * NPOINT + s0, S_PER_W)], cyv.at[pl.ds(0, S_PER_W)])
    pltpu.sync_copy(cb_hbm.at[pl.ds((b * 3 + 2) * NPOINT + s0, S_PER_W)], czv.at[pl.ds(0, S_PER_W)])
    pltpu.sync_copy(s2_hbm.at[pl.ds(b * NPOINT + s0, S_PER_W)], s2v.at[pl.ds(0, S_PER_W)])

    lane = lax.iota(jnp.int32, 16)
    ones = jnp.ones((16,), jnp.int32)
    zeros = jnp.zeros((16,), jnp.int32)

    def per_centroid(s, _):
        cx = cxv[pl.ds(s, 16)][0]
        cy = cyv[pl.ds(s, 16)][0]
        cz = czv[pl.ds(s, 16)][0]
        s2 = s2v[pl.ds(s, 16)][0]

        def chunk_mask(base):
            m1 = xv[pl.ds(base, 16)] * cx
            m2 = yv[pl.ds(base, 16)] * cy
            m3 = zv[pl.ds(base, 16)] * cz
            q1 = (m1 * SCALE).astype(jnp.int32)
            q2 = (m2 * SCALE).astype(jnp.int32)
            q3 = (m3 * SCALE).astype(jnp.int32)
            mm = ((q1 + q2) + q3).astype(jnp.float32) * INV_SCALE
            sq = (s2 + d2v[pl.ds(base, 16)]) - 2.0 * mm
            return jnp.logical_not(sq > R2)

        def cond(carry):
            ck, cnt = carry
            return (cnt < NSAMPLE) & (ck < NCHUNK // 2)

        def body(carry):
            ck, cnt = carry
            base = ck * 32
            mask0 = chunk_mask(base)
            mask1 = chunk_mask(base + 16)
            t0 = plsc.all_reduce_population_count(mask0)[0]
            t1 = plsc.all_reduce_population_count(mask1)[0]
            inc0 = plsc.cumsum(jnp.where(mask0, ones, zeros))
            plsc.store_scatter(idxbuf, [jnp.minimum(cnt + inc0 - 1, 63)],
                               base + lane, mask=mask0)
            inc1 = plsc.cumsum(jnp.where(mask1, ones, zeros))
            plsc.store_scatter(idxbuf, [jnp.minimum(cnt + t0 + inc1 - 1, 63)],
                               base + 16 + lane, mask=mask1)
            return ck + 1, cnt + (t0 + t1)

        _, cnt = lax.while_loop(cond, body, (0, 0))
        first = idxbuf[pl.ds(0, 16)][0]
        goff = b * N
        full = jnp.full((16,), True, jnp.bool_)
        for h in range(2):
            ids = idxbuf[pl.ds(h * 16, 16)]
            ids = jnp.where(lane + h * 16 < cnt, ids, first)
            plsc.store_scatter(gidx, [s * NSAMPLE + h * 16 + lane], ids + goff, mask=full)
        return 0

    lax.fori_loop(0, S_PER_W, per_centroid, 0)

    row0 = (b * NPOINT + s0) * NSAMPLE

    def gather_grp(j, _):
        g0 = j * 128
        for t in range(8):
            gv = gidx[pl.ds(g0 + t * 16, 16)]
            plsc.store_scatter(srow, [t * 16 + lane], gv // 4, mask=jnp.full((16,), True, jnp.bool_))
        pltpu.async_copy(table_hbm.at[srow], rows, sem).wait()

        def extract(r, _):
            off = (gidx[pl.ds(g0 + r, 16)][0] % 4) * CH_PAD
            packed[pl.ds(r * CH_PAD, 16)] = rows[r, pl.ds(off, 16)]
            packed[pl.ds(r * CH_PAD + 16, 16)] = rows[r, pl.ds(off + 16, 16)]
            return 0

        lax.fori_loop(0, 128, extract, 0)
        pltpu.sync_copy(packed, out_hbm.at[pl.ds((row0 + g0) * CH_PAD, 128 * CH_PAD)])
        return 0

    lax.fori_loop(0, (S_PER_W * NSAMPLE) // 128, gather_grp, 0)


def _ball_query_gather(xyz, points, new_xyz):
    xt = xyz.transpose(0, 2, 1)
    ct = new_xyz.transpose(0, 2, 1)
    dst2 = jnp.sum(xt ** 2, -1)
    src2 = jnp.sum(ct ** 2, -1)
    xb = lax.optimization_barrier(xyz.astype(jnp.bfloat16)).astype(jnp.float32)
    cb = lax.optimization_barrier(new_xyz.astype(jnp.bfloat16)).astype(jnp.float32)
    table = jnp.concatenate(
        [xt, points.transpose(0, 2, 1),
         jnp.zeros((B, N, CH_PAD - IN_CH), jnp.float32)], axis=-1).reshape(B * N, CH_PAD)

    grouped = pl.kernel(
        _bq_sc_kernel,
        out_type=jax.ShapeDtypeStruct((B * NPOINT * NSAMPLE * CH_PAD,), jnp.float32),
        mesh=plsc.VectorSubcoreMesh(core_axis_name="c", subcore_axis_name="s"),
        compiler_params=pltpu.CompilerParams(needs_layout_passes=False),
        scratch_types=[
            pltpu.VMEM((N,), jnp.float32),
            pltpu.VMEM((N,), jnp.float32),
            pltpu.VMEM((N,), jnp.float32),
            pltpu.VMEM((N,), jnp.float32),
            pltpu.VMEM((S_PER_W + 16,), jnp.float32),
            pltpu.VMEM((S_PER_W + 16,), jnp.float32),
            pltpu.VMEM((S_PER_W + 16,), jnp.float32),
            pltpu.VMEM((S_PER_W + 16,), jnp.float32),
            pltpu.VMEM((64,), jnp.int32),
            pltpu.VMEM((S_PER_W * NSAMPLE + 16,), jnp.int32),
            pltpu.VMEM((128,), jnp.int32),
            pltpu.VMEM((128, 128), jnp.float32),
            pltpu.VMEM((128 * CH_PAD,), jnp.float32),
            pltpu.SemaphoreType.DMA,
        ],
    )(xb.reshape(-1), cb.reshape(-1), dst2.reshape(-1), src2.reshape(-1),
      table.reshape(B * N // 4, 4 * CH_PAD))
    return grouped.reshape(B, NPOINT, NSAMPLE, CH_PAD)


def _mlp_kernel(g_ref, sub_ref, w1_ref, w2_ref, w3_ref,
                p1_ref, p2_ref, p3_ref, out_ref):
    g = g_ref[0]
    sub = sub_ref[0]
    x = (g - sub[:, None, :]).reshape(S_BLK * NSAMPLE, CH_PAD)

    def layer(h, w_ref, p_ref):
        mm = jnp.dot(h.astype(jnp.bfloat16), w_ref[...],
                     preferred_element_type=jnp.float32)
        p = p_ref[...]
        hh = (mm + p[0:1] - p[1:2]) / p[2:3] * p[3:4] + p[4:5]
        return jnp.maximum(hh, 0.0)

    h1 = layer(x, w1_ref, p1_ref)
    h2 = layer(h1, w2_ref, p2_ref)
    h3 = layer(h2, w3_ref, p3_ref)
    out_ref[0] = jnp.max(h3.reshape(S_BLK, NSAMPLE, 64), axis=1)


def _mlp_max(grouped, sub, weights, stats):
    w1, w2, w3 = weights
    p1, p2, p3 = stats
    grid = (B, NPOINT // S_BLK)
    return pl.pallas_call(
        _mlp_kernel,
        grid=grid,
        in_specs=[
            pl.BlockSpec((1, S_BLK, NSAMPLE, CH_PAD), lambda b, s: (b, s, 0, 0)),
            pl.BlockSpec((1, S_BLK, CH_PAD), lambda b, s: (b, s, 0)),
            pl.BlockSpec(w1.shape, lambda b, s: (0, 0)),
            pl.BlockSpec(w2.shape, lambda b, s: (0, 0)),
            pl.BlockSpec(w3.shape, lambda b, s: (0, 0)),
            pl.BlockSpec(p1.shape, lambda b, s: (0, 0)),
            pl.BlockSpec(p2.shape, lambda b, s: (0, 0)),
            pl.BlockSpec(p3.shape, lambda b, s: (0, 0)),
        ],
        out_specs=pl.BlockSpec((1, S_BLK, 64), lambda b, s: (b, s, 0)),
        out_shape=jax.ShapeDtypeStruct((B, NPOINT, 64), jnp.float32),
    )(grouped, sub, w1, w2, w3, p1, p2, p3)


def kernel(xyz, points, conv_w0, conv_b0, bn_g0, bn_b0, bn_rm0, bn_rv0,
           conv_w1, conv_b1, bn_g1, bn_b1, bn_rm1, bn_rv1,
           conv_w2, conv_b2, bn_g2, bn_b2, bn_rm2, bn_rv2):
    new_xyz = _fps(xyz)
    grouped = _ball_query_gather(xyz, points, new_xyz)
    sub = jnp.concatenate(
        [new_xyz.transpose(0, 2, 1),
         jnp.zeros((B, NPOINT, CH_PAD - 3), jnp.float32)], axis=-1)

    def prep_w(w, in_ch):
        wm = w[:, :, 0, 0].T
        pad = jnp.zeros((CH_PAD - in_ch, wm.shape[1]), wm.dtype) if wm.shape[0] < CH_PAD else None
        if pad is not None:
            wm = jnp.concatenate([wm, pad], axis=0)
        return wm.astype(jnp.bfloat16)

    w1 = prep_w(conv_w0, IN_CH)
    w2 = conv_w1[:, :, 0, 0].T.astype(jnp.bfloat16)
    w3 = conv_w2[:, :, 0, 0].T.astype(jnp.bfloat16)
    p1 = jnp.stack([conv_b0, bn_rm0, jnp.sqrt(bn_rv0 + EPS), bn_g0, bn_b0])
    p2 = jnp.stack([conv_b1, bn_rm1, jnp.sqrt(bn_rv1 + EPS), bn_g1, bn_b1])
    p3 = jnp.stack([conv_b2, bn_rm2, jnp.sqrt(bn_rv2 + EPS), bn_g2, bn_b2])

    out = _mlp_max(grouped, sub, (w1, w2, w3), (p1, p2, p3))
    return new_xyz, out.transpose(0, 2, 1)
